# Initial kernel scaffold; baseline (speedup 1.0000x reference)
import jax
import jax.numpy as jnp
from jax.experimental import pallas as pl


def kernel(h_V, h_E, E_idx, W1_w, W1_b, W2_w, W2_b, W3_w, W3_b, W11_w, W11_b, W12_w, W12_b, W13_w, W13_b, Win_w, Win_b, Wout_w, Wout_b, ln1_g, ln1_b, ln2_g, ln2_b, ln3_g, ln3_b):
    raise NotImplementedError("write your pallas kernel here")


# speedup vs baseline: 13.0321x; 13.0321x over previous
import functools

import jax
import jax.numpy as jnp
from jax import lax
from jax.experimental import pallas as pl
from jax.experimental.pallas import tpu as pltpu
from jax.experimental.pallas import tpu_sc as plsc

B, L, K, H, NIN = 4, 2048, 32, 128, 256
SCALE = 30.0
NB = 128
F32 = jnp.float32

_SQRT_HALF = 0.7071067811865476


def _gelu(x):
    return 0.5 * x * (1.0 + lax.erf(x * _SQRT_HALF))


def _ln(x, g, b, eps=1e-5):
    mu = jnp.mean(x, axis=-1, keepdims=True)
    xc = x - mu
    var = jnp.mean(xc * xc, axis=-1, keepdims=True)
    return xc * lax.rsqrt(var + eps) * g + b


def _sc_gather(table, idx):
    N = idx.shape[0]
    info = plsc.get_sparse_core_info()
    NC, NS, LN = info.num_cores, info.num_subcores, info.num_lanes
    NW = NC * NS
    per_w = N // NW
    CH = 128
    n_ch = per_w // CH
    per_batch = L * K

    mesh = plsc.VectorSubcoreMesh(core_axis_name="c", subcore_axis_name="s")

    @functools.partial(
        pl.kernel,
        mesh=mesh,
        out_type=jax.ShapeDtypeStruct((N, H), F32),
        scratch_types=[
            pltpu.VMEM((CH,), jnp.int32),
            pltpu.VMEM((CH, H), F32),
            pltpu.SemaphoreType.DMA,
        ],
    )
    def k(table_hbm, idx_hbm, out_hbm, idx_v, rows_v, sem):
        wid = lax.axis_index("s") * NC + lax.axis_index("c")
        base = wid * per_w

        def body(i, carry):
            off = base + i * CH
            pltpu.sync_copy(idx_hbm.at[pl.ds(off, CH)], idx_v)
            boff = (off // per_batch) * L

            def adj(j, c):
                sl = pl.ds(j * LN, LN)
                idx_v[sl] = idx_v[sl] + boff
                return c

            lax.fori_loop(0, CH // LN, adj, 0)
            pltpu.async_copy(table_hbm.at[idx_v], rows_v, sem).wait()
            pltpu.sync_copy(rows_v, out_hbm.at[pl.ds(off, CH)])
            return carry

        lax.fori_loop(0, n_ch, body, 0)

    return k(table, idx)


def _pre_body(hv_ref, w1a_ref, w1c_ref, b1_ref, p1_ref, s1_ref):
    hv = hv_ref[0]
    p1_ref[0] = jnp.dot(hv, w1c_ref[...], preferred_element_type=F32)
    s1_ref[0] = jnp.dot(hv, w1a_ref[...], preferred_element_type=F32) + b1_ref[...]


def _pre(h_V, W1a, W1c, b1):
    return pl.pallas_call(
        _pre_body,
        grid=(B,),
        in_specs=[
            pl.BlockSpec((1, L, H), lambda b: (b, 0, 0)),
            pl.BlockSpec((H, H), lambda b: (0, 0)),
            pl.BlockSpec((H, H), lambda b: (0, 0)),
            pl.BlockSpec((1, H), lambda b: (0, 0)),
        ],
        out_specs=[
            pl.BlockSpec((1, L, H), lambda b: (b, 0, 0)),
            pl.BlockSpec((1, L, H), lambda b: (b, 0, 0)),
        ],
        out_shape=[
            jax.ShapeDtypeStruct((B, L, H), F32),
            jax.ShapeDtypeStruct((B, L, H), F32),
        ],
    )(h_V, W1a, W1c, b1)


def _edge1_body(hE_ref, g1_ref, s1_ref, hv_ref,
                w1b_ref, w2_ref, b2_ref, w3_ref, b3_ref,
                win_ref, bin_ref, wout_ref, bout_ref,
                ln1g_ref, ln1b_ref, ln2g_ref, ln2b_ref,
                w11a_ref, w11c_ref, b11_ref,
                v2_ref, p2_ref, s2_ref):
    NBK = NB * K
    x = hE_ref[0].reshape(NBK, H)
    g = g1_ref[0].reshape(NBK, H)
    s1 = s1_ref[0]
    a = jnp.dot(x, w1b_ref[...], preferred_element_type=F32) + g
    a = a + jnp.broadcast_to(s1[:, None, :], (NB, K, H)).reshape(NBK, H)
    h = _gelu(a)
    h = _gelu(jnp.dot(h, w2_ref[...], preferred_element_type=F32) + b2_ref[...])
    m = jnp.dot(h, w3_ref[...], preferred_element_type=F32) + b3_ref[...]
    dh = jnp.sum(m.reshape(NB, K, H), axis=1) * (1.0 / SCALE)
    v1 = _ln(hv_ref[0] + dh, ln1g_ref[...], ln1b_ref[...])
    ff = jnp.dot(_gelu(jnp.dot(v1, win_ref[...], preferred_element_type=F32)
                       + bin_ref[...]),
                 wout_ref[...], preferred_element_type=F32) + bout_ref[...]
    v2 = _ln(v1 + ff, ln2g_ref[...], ln2b_ref[...])
    v2_ref[0] = v2
    p2_ref[0] = jnp.dot(v2, w11c_ref[...], preferred_element_type=F32)
    s2_ref[0] = jnp.dot(v2, w11a_ref[...], preferred_element_type=F32) + b11_ref[...]


def _edge1(h_E, g1, s1, h_V, W1b, W2_w, W2_b, W3_w, W3_b,
           Win_w, Win_b, Wout_w, Wout_b, ln1_g, ln1_b, ln2_g, ln2_b,
           W11a, W11c, b11):
    wspec = lambda r, c: pl.BlockSpec((r, c), lambda b, i: (0, 0))
    return pl.pallas_call(
        _edge1_body,
        grid=(B, L // NB),
        in_specs=[
            pl.BlockSpec((1, NB, K, H), lambda b, i: (b, i, 0, 0)),
            pl.BlockSpec((1, NB, K, H), lambda b, i: (b, i, 0, 0)),
            pl.BlockSpec((1, NB, H), lambda b, i: (b, i, 0)),
            pl.BlockSpec((1, NB, H), lambda b, i: (b, i, 0)),
            wspec(H, H), wspec(H, H), wspec(1, H), wspec(H, H), wspec(1, H),
            wspec(H, 4 * H), wspec(1, 4 * H), wspec(4 * H, H), wspec(1, H),
            wspec(1, H), wspec(1, H), wspec(1, H), wspec(1, H),
            wspec(H, H), wspec(H, H), wspec(1, H),
        ],
        out_specs=[
            pl.BlockSpec((1, NB, H), lambda b, i: (b, i, 0)),
            pl.BlockSpec((1, NB, H), lambda b, i: (b, i, 0)),
            pl.BlockSpec((1, NB, H), lambda b, i: (b, i, 0)),
        ],
        out_shape=[
            jax.ShapeDtypeStruct((B, L, H), F32),
            jax.ShapeDtypeStruct((B, L, H), F32),
            jax.ShapeDtypeStruct((B, L, H), F32),
        ],
    )(h_E, g1, s1, h_V, W1b, W2_w, W2_b, W3_w, W3_b,
      Win_w, Win_b, Wout_w, Wout_b, ln1_g, ln1_b, ln2_g, ln2_b,
      W11a, W11c, b11)


def _edge2_body(hE_ref, g2_ref, s2_ref,
                w11b_ref, w12_ref, b12_ref, w13_ref, b13_ref,
                ln3g_ref, ln3b_ref, out_ref):
    NBK = NB * K
    x = hE_ref[0].reshape(NBK, H)
    g = g2_ref[0].reshape(NBK, H)
    s2 = s2_ref[0]
    a = jnp.dot(x, w11b_ref[...], preferred_element_type=F32) + g
    a = a + jnp.broadcast_to(s2[:, None, :], (NB, K, H)).reshape(NBK, H)
    h = _gelu(a)
    h = _gelu(jnp.dot(h, w12_ref[...], preferred_element_type=F32) + b12_ref[...])
    m = jnp.dot(h, w13_ref[...], preferred_element_type=F32) + b13_ref[...]
    e = _ln(x + m, ln3g_ref[...], ln3b_ref[...])
    out_ref[0] = e.reshape(NB, K, H)


def _edge2(h_E, g2, s2, W11b, W12_w, W12_b, W13_w, W13_b, ln3_g, ln3_b):
    wspec = lambda r, c: pl.BlockSpec((r, c), lambda b, i: (0, 0))
    return pl.pallas_call(
        _edge2_body,
        grid=(B, L // NB),
        in_specs=[
            pl.BlockSpec((1, NB, K, H), lambda b, i: (b, i, 0, 0)),
            pl.BlockSpec((1, NB, K, H), lambda b, i: (b, i, 0, 0)),
            pl.BlockSpec((1, NB, H), lambda b, i: (b, i, 0)),
            wspec(H, H), wspec(H, H), wspec(1, H), wspec(H, H), wspec(1, H),
            wspec(1, H), wspec(1, H),
        ],
        out_specs=[pl.BlockSpec((1, NB, K, H), lambda b, i: (b, i, 0, 0))],
        out_shape=[jax.ShapeDtypeStruct((B, L, K, H), F32)],
    )(h_E, g2, s2, W11b, W12_w, W12_b, W13_w, W13_b, ln3_g, ln3_b)


def kernel(h_V, h_E, E_idx, W1_w, W1_b, W2_w, W2_b, W3_w, W3_b,
           W11_w, W11_b, W12_w, W12_b, W13_w, W13_b, Win_w, Win_b,
           Wout_w, Wout_b, ln1_g, ln1_b, ln2_g, ln2_b, ln3_g, ln3_b):
    r1 = lambda v: v.reshape(1, -1)
    idx = E_idx.reshape(-1).astype(jnp.int32)

    W1a, W1b, W1c = W1_w[:H], W1_w[H:H + H], W1_w[H + H:]
    W11a, W11b, W11c = W11_w[:H], W11_w[H:H + H], W11_w[H + H:]

    P1, s1 = _pre(h_V, W1a, W1c, r1(W1_b))
    g1 = _sc_gather(P1.reshape(B * L, H), idx).reshape(B, L, K, H)
    v2, P2, s2 = _edge1(
        h_E, g1, s1, h_V, W1b, W2_w, r1(W2_b), W3_w, r1(W3_b),
        Win_w, r1(Win_b), Wout_w, r1(Wout_b),
        r1(ln1_g), r1(ln1_b), r1(ln2_g), r1(ln2_b),
        W11a, W11c, r1(W11_b))
    g2 = _sc_gather(P2.reshape(B * L, H), idx).reshape(B, L, K, H)
    (hE_out,) = _edge2(
        h_E, g2, s2, W11b, W12_w, r1(W12_b), W13_w, r1(W13_b),
        r1(ln3_g), r1(ln3_b))
    return (v2, hE_out)
